# baseline (speedup 1.0000x reference)
import functools

import jax
import jax.numpy as jnp
from jax import lax
from jax.experimental import pallas as pl
from jax.experimental.pallas import tpu as pltpu
from jax.experimental.pallas import tpu_sc as plsc

B = 4096
D = 2048
A = 64
H = 1024
BASE = 5

BLK = 128
NBLK = 36
RS = NBLK * BLK
ARP = 128

NC, NS = 2, 16
NW = NC * NS
RPW = RS // NW
CH = 24
NCHH = RPW // CH

_f32 = jnp.float32
_bf16 = jnp.bfloat16


def _routing(labels):
    li = labels.astype(jnp.int32)
    order = jnp.argsort(li).astype(jnp.int32)
    counts = jnp.bincount(li, length=BASE).astype(jnp.int32)
    starts = jnp.concatenate(
        [jnp.zeros((1,), jnp.int32), jnp.cumsum(counts)[:-1].astype(jnp.int32)])
    nb = (counts + BLK - 1) // BLK
    boff = jnp.concatenate(
        [jnp.zeros((1,), jnp.int32), jnp.cumsum(nb)[:-1].astype(jnp.int32)])
    total_nb = jnp.sum(nb)
    j = jnp.arange(NBLK, dtype=jnp.int32)
    e_j = jnp.clip(jnp.searchsorted(boff, j, side='right').astype(jnp.int32) - 1,
                   0, BASE - 1)
    t_j = j - boff[e_j]
    lastpos = starts[e_j] + counts[e_j] - 1
    pos = starts[e_j][:, None] + t_j[:, None] * BLK + jnp.arange(BLK, dtype=jnp.int32)[None, :]
    pos = jnp.clip(jnp.minimum(pos, lastpos[:, None]), 0, B - 1)
    row_ids = order[pos]
    valid = j < total_nb
    be = jnp.where(valid, e_j, li[0]).astype(jnp.int32)
    row_ids = jnp.where(valid[:, None], row_ids, 0).astype(jnp.int32)
    return be, row_ids.reshape(-1)


def _sc_mesh():
    return plsc.VectorSubcoreMesh(core_axis_name="c", subcore_axis_name="s")


@functools.lru_cache(maxsize=None)
def _gather_kernel():
    @functools.partial(
        pl.kernel,
        mesh=_sc_mesh(),
        out_type=[jax.ShapeDtypeStruct((RS, D), _f32),
                  jax.ShapeDtypeStruct((RS, D), _f32),
                  jax.ShapeDtypeStruct((RS, ARP), _f32)],
        scratch_types=[pltpu.VMEM((RPW // 2,), jnp.int32),
                       pltpu.VMEM((RPW // 2, ARP), _f32),
                       pltpu.VMEM((CH,), jnp.int32),
                       pltpu.VMEM((CH,), jnp.int32),
                       pltpu.VMEM((CH, D), _f32),
                       pltpu.VMEM((CH, D), _f32),
                       pltpu.SemaphoreType.DMA,
                       pltpu.SemaphoreType.DMA,
                       pltpu.SemaphoreType.DMA],
    )
    def _sc_gather(s1_hbm, s2_hbm, ar_hbm, idx_hbm, o1, o2, o3,
                   ia, ba, i0, i1, b0, b1, sem0, sem1, sema):
        wid = lax.axis_index("s") * NC + lax.axis_index("c")
        base = wid * RPW
        HA = RPW // 2
        pltpu.sync_copy(idx_hbm.at[pl.ds(base, HA)], ia)
        ar_cp = pltpu.async_copy(ar_hbm.at[ia], ba, sema)

        idxs = (i0, i1)
        bufs = (b0, b1)
        sems = (sem0, sem1)
        NTOT = 2 * NCHH

        def src_dst_off(ci):
            if ci < NCHH:
                return s1_hbm, o1, base + ci * CH
            return s2_hbm, o2, base + (ci - NCHH) * CH

        def fire(ci):
            sl = ci % 2
            src, _, off = src_dst_off(ci)
            pltpu.sync_copy(idx_hbm.at[pl.ds(off, CH)], idxs[sl])
            return pltpu.async_copy(src.at[idxs[sl]], bufs[sl], sems[sl])

        pend = [fire(0), fire(1)]
        for ci in range(NTOT):
            sl = ci % 2
            _, dst, off = src_dst_off(ci)
            pend[sl].wait()
            pltpu.sync_copy(bufs[sl], dst.at[pl.ds(off, CH)])
            if ci + 2 < NTOT:
                pend[sl] = fire(ci + 2)

        ar_cp.wait()
        pltpu.sync_copy(ba, o3.at[pl.ds(base, HA)])
        pltpu.sync_copy(idx_hbm.at[pl.ds(base + HA, HA)], ia)
        pltpu.async_copy(ar_hbm.at[ia], ba, sema).wait()
        pltpu.sync_copy(ba, o3.at[pl.ds(base + HA, HA)])

    return _sc_gather


@functools.lru_cache(maxsize=None)
def _scatter_kernel():
    @functools.partial(
        pl.kernel,
        mesh=_sc_mesh(),
        out_type=[jax.ShapeDtypeStruct((B, D), _f32),
                  jax.ShapeDtypeStruct((B, D), _f32)],
        scratch_types=[pltpu.VMEM((CH,), jnp.int32),
                       pltpu.VMEM((CH,), jnp.int32),
                       pltpu.VMEM((CH, D), _f32),
                       pltpu.VMEM((CH, D), _f32),
                       pltpu.SemaphoreType.DMA,
                       pltpu.SemaphoreType.DMA],
    )
    def _sc_scatter(mu_s_hbm, lv_s_hbm, idx_hbm, mu_o, lv_o,
                    i0, i1, b0, b1, sem0, sem1):
        wid = lax.axis_index("s") * NC + lax.axis_index("c")
        base = wid * RPW
        idxs = (i0, i1)
        bufs = (b0, b1)
        sems = (sem0, sem1)
        NTOT = 2 * NCHH

        def src_dst_off(ci):
            if ci < NCHH:
                return mu_s_hbm, mu_o, base + ci * CH
            return lv_s_hbm, lv_o, base + (ci - NCHH) * CH

        pend = {}
        for ci in range(NTOT):
            sl = ci % 2
            src, dst, off = src_dst_off(ci)
            if sl in pend:
                pend[sl].wait()
            pltpu.sync_copy(idx_hbm.at[pl.ds(off, CH)], idxs[sl])
            pltpu.sync_copy(src.at[pl.ds(off, CH)], bufs[sl])
            pend[sl] = pltpu.async_copy(bufs[sl], dst.at[idxs[sl]], sems[sl])
        for c in pend.values():
            c.wait()

    return _sc_scatter


def _dot(x, w):
    return lax.dot_general(x, w, (((1,), (0,)), ((), ())),
                           preferred_element_type=_f32)


def _enc_front_body(be_ref, s1, s2, ar,
                    se1W, se1b, se2W, se2b, nse1W, nse1b, nse2W, nse2b,
                    Wh1, Wh2, AEp, REv, bp, ce2W, ce2b, c2_o):
    t1 = jnp.maximum(_dot(s1[...].astype(_bf16), se1W[0]) + se1b[0],
                     0.0).astype(_bf16)
    h1 = (_dot(t1, se2W[0]) + se2b[0]).astype(_bf16)
    t2 = jnp.maximum(_dot(s2[...].astype(_bf16), nse1W[0]) + nse1b[0],
                     0.0).astype(_bf16)
    h2 = (_dot(t2, nse2W[0]) + nse2b[0]).astype(_bf16)
    ab = ar[:, :A].astype(_bf16)
    rb = ar[:, A:A + 1]
    cpre = (_dot(h1, Wh1[0]) + _dot(h2, Wh2[0]) + _dot(ab, AEp[0])
            + rb * REv[0] + bp[0])
    c2_o[...] = (_dot(jnp.maximum(cpre, 0.0).astype(_bf16), ce2W[0])
                 + ce2b[0]).astype(_bf16)


def _enc_back_body(be_ref, c2, muW, mub, lvW, lvb, mu_o, lv_o):
    c2b = c2[...]
    mu_o[...] = _dot(c2b, muW[0]) + mub[0]
    lv_o[...] = _dot(c2b, lvW[0]) + lvb[0]


def _softmax(x):
    m = jnp.max(x, axis=-1, keepdims=True)
    e = jnp.exp(x - m)
    return e / jnp.sum(e, axis=-1, keepdims=True)


def _dec_body(mu, lv, eps,
              sd1W, sd1b, sd2W, sd2b, sd3W, sd3b,
              nsd1W, nsd1b, nsd2W, nsd2b, nsd3W, nsd3b,
              ad1W, ad1b, ad2W, ad2b,
              sd_o, nsd_o, ad_o, z_o):
    z = mu[...] + eps[...] * jnp.exp(0.5 * lv[...])
    z_o[...] = z
    zb = z.astype(_bf16)

    def mlp3(w1, b1, w2, b2, w3, b3):
        h = jnp.maximum(_dot(zb, w1[...]) + b1[0], 0.0).astype(_bf16)
        h = jnp.maximum(_dot(h, w2[...]) + b2[0], 0.0).astype(_bf16)
        return _softmax(_dot(h, w3[...]) + b3[0])

    sd_o[...] = mlp3(sd1W, sd1b, sd2W, sd2b, sd3W, sd3b)
    nsd_o[...] = mlp3(nsd1W, nsd1b, nsd2W, nsd2b, nsd3W, nsd3b)
    ha = jnp.maximum(_dot(zb, ad1W[...]) + ad1b[0], 0.0).astype(_bf16)
    ad_o[...] = _softmax(_dot(ha, ad2W[...]) + ad2b[0])


def kernel(s_t_1, a, r, s_t_2, labels, params):
    be, flat_ids = _routing(labels)

    ar = jnp.concatenate(
        [a, r, jnp.zeros((B, ARP - A - 1), _f32)], axis=1)

    s1_s, s2_s, ar_s = _gather_kernel()(s_t_1, s_t_2, ar, flat_ids)

    enc = params['enc']
    def stkw(xs):
        return jnp.stack([x.astype(_bf16) for x in xs])
    def stkb(xs):
        return jnp.stack([x.reshape(1, -1) for x in xs])
    se1W, se1b = stkw([e['se1'][0] for e in enc]), stkb([e['se1'][1] for e in enc])
    se2W, se2b = stkw([e['se2'][0] for e in enc]), stkb([e['se2'][1] for e in enc])
    nse1W, nse1b = stkw([e['nse1'][0] for e in enc]), stkb([e['nse1'][1] for e in enc])
    nse2W, nse2b = stkw([e['nse2'][0] for e in enc]), stkb([e['nse2'][1] for e in enc])
    ce2W, ce2b = stkw([e['ce2'][0] for e in enc]), stkb([e['ce2'][1] for e in enc])
    muW, mub = stkw([e['mu'][0] for e in enc]), stkb([e['mu'][1] for e in enc])
    lvW, lvb = stkw([e['lv'][0] for e in enc]), stkb([e['lv'][1] for e in enc])

    Wh1 = stkw([e['ce1'][0][:H] for e in enc])
    Wh2 = stkw([e['ce1'][0][H + 12:] for e in enc])
    AEp = stkw([e['ae'][0] @ e['ce1'][0][H:H + 8] for e in enc])
    REv = stkb([e['re'][0] @ e['ce1'][0][H + 8:H + 12] for e in enc])
    bp = stkb([e['ce1'][1] + e['ae'][1] @ e['ce1'][0][H:H + 8]
               + e['re'][1] @ e['ce1'][0][H + 8:H + 12] for e in enc])

    wspec3 = lambda d1, d2: pl.BlockSpec((1, d1, d2), lambda j, be: (be[j], 0, 0))
    wspec2 = lambda d1: pl.BlockSpec((1, 1, d1), lambda j, be: (be[j], 0, 0))
    rspec = lambda d1: pl.BlockSpec((BLK, d1), lambda j, be: (j, 0))

    front_spec = pltpu.PrefetchScalarGridSpec(
        num_scalar_prefetch=1,
        grid=(NBLK,),
        in_specs=[
            rspec(D), rspec(D), rspec(ARP),
            wspec3(D, 128), wspec2(128), wspec3(128, H), wspec2(H),
            wspec3(D, 128), wspec2(128), wspec3(128, H), wspec2(H),
            wspec3(H, H), wspec3(H, H), wspec3(A, H), wspec2(H), wspec2(H),
            wspec3(H, H), wspec2(H),
        ],
        out_specs=[rspec(H)],
    )
    (c2_s,) = pl.pallas_call(
        _enc_front_body,
        grid_spec=front_spec,
        out_shape=[jax.ShapeDtypeStruct((RS, H), _bf16)],
        compiler_params=pltpu.CompilerParams(
            dimension_semantics=("arbitrary",)),
    )(be, s1_s, s2_s, ar_s,
      se1W, se1b, se2W, se2b, nse1W, nse1b, nse2W, nse2b,
      Wh1, Wh2, AEp, REv, bp, ce2W, ce2b)

    back_spec = pltpu.PrefetchScalarGridSpec(
        num_scalar_prefetch=1,
        grid=(NBLK,),
        in_specs=[
            rspec(H),
            wspec3(H, D), wspec2(D), wspec3(H, D), wspec2(D),
        ],
        out_specs=[rspec(D), rspec(D)],
    )
    mu_s, lv_s = pl.pallas_call(
        _enc_back_body,
        grid_spec=back_spec,
        out_shape=[jax.ShapeDtypeStruct((RS, D), _f32),
                   jax.ShapeDtypeStruct((RS, D), _f32)],
        compiler_params=pltpu.CompilerParams(
            dimension_semantics=("arbitrary",)),
    )(be, c2_s, muW, mub, lvW, lvb)

    mu, lv = _scatter_kernel()(mu_s, lv_s, flat_ids)

    try:
        with jax.ensure_compile_time_eval():
            eps = jax.random.normal(jax.random.key(42), (B, D), dtype=_f32)
    except Exception:
        eps = jax.random.normal(jax.random.key(42), (B, D), dtype=_f32)

    RB = 256
    row = lambda d1: pl.BlockSpec((RB, d1), lambda j: (j, 0))
    w2 = lambda d1, d2: pl.BlockSpec((d1, d2), lambda j: (0, 0))
    bvec = lambda d1: pl.BlockSpec((1, d1), lambda j: (0, 0))
    p = params
    sd, nsd, ad, z = pl.pallas_call(
        _dec_body,
        grid=(B // RB,),
        in_specs=[
            row(D), row(D), row(D),
            w2(D, 128), bvec(128), w2(128, 128), bvec(128), w2(128, D), bvec(D),
            w2(D, 128), bvec(128), w2(128, 128), bvec(128), w2(128, D), bvec(D),
            w2(D, H), bvec(H), w2(H, A), bvec(A),
        ],
        out_specs=[row(D), row(D), row(A), row(D)],
        out_shape=[jax.ShapeDtypeStruct((B, D), _f32),
                   jax.ShapeDtypeStruct((B, D), _f32),
                   jax.ShapeDtypeStruct((B, A), _f32),
                   jax.ShapeDtypeStruct((B, D), _f32)],
        compiler_params=pltpu.CompilerParams(
            dimension_semantics=("arbitrary",)),
    )(mu, lv, eps,
      p['sd1'][0].astype(_bf16), p['sd1'][1].reshape(1, -1),
      p['sd2'][0].astype(_bf16), p['sd2'][1].reshape(1, -1),
      p['sd3'][0].astype(_bf16), p['sd3'][1].reshape(1, -1),
      p['nsd1'][0].astype(_bf16), p['nsd1'][1].reshape(1, -1),
      p['nsd2'][0].astype(_bf16), p['nsd2'][1].reshape(1, -1),
      p['nsd3'][0].astype(_bf16), p['nsd3'][1].reshape(1, -1),
      p['ad1'][0].astype(_bf16), p['ad1'][1].reshape(1, -1),
      p['ad2'][0].astype(_bf16), p['ad2'][1].reshape(1, -1))

    rd = jnp.ones((B, 1), _f32)

    return (sd, ad, rd, nsd, mu, lv, z)


# speedup vs baseline: 1.0555x; 1.0555x over previous
import functools

import jax
import jax.numpy as jnp
from jax import lax
from jax.experimental import pallas as pl
from jax.experimental.pallas import tpu as pltpu
from jax.experimental.pallas import tpu_sc as plsc

B = 4096
D = 2048
A = 64
H = 1024
BASE = 5

BLK = 128
NBLK = 36
RS = NBLK * BLK
ARP = 128

NC, NS = 2, 16
NW = NC * NS
RPW = RS // NW
CH = 24
NCHH = RPW // CH

_f32 = jnp.float32
_bf16 = jnp.bfloat16


def _routing(labels):
    li = labels.astype(jnp.int32)
    iota = jnp.arange(B, dtype=jnp.int32)
    sl, order = lax.sort((li, iota), num_keys=1)
    ar5 = jnp.arange(BASE, dtype=jnp.int32)
    counts = jnp.sum((li[None, :] == ar5[:, None]).astype(jnp.int32), axis=1)
    starts = jnp.concatenate(
        [jnp.zeros((1,), jnp.int32), jnp.cumsum(counts)[:-1].astype(jnp.int32)])
    nb = (counts + BLK - 1) // BLK
    boff = jnp.concatenate(
        [jnp.zeros((1,), jnp.int32), jnp.cumsum(nb)[:-1].astype(jnp.int32)])
    total_nb = jnp.sum(nb)
    j = jnp.arange(NBLK, dtype=jnp.int32)
    e_j = jnp.clip(
        jnp.sum((boff[None, :] <= j[:, None]).astype(jnp.int32), axis=1) - 1,
        0, BASE - 1)
    valid = j < total_nb
    be = jnp.where(valid, e_j, li[0]).astype(jnp.int32)
    m5 = (sl[:, None] == ar5[None, :]).astype(jnp.int32)
    bo_i = jnp.sum(m5 * boff[None, :], axis=1)
    st_i = jnp.sum(m5 * starts[None, :], axis=1)
    dst = bo_i * BLK + iota - st_i
    flat0 = jnp.zeros((RS,), jnp.int32).at[dst].set(order + 1)
    f2 = flat0.reshape(NBLK, BLK)
    bm = jnp.max(f2, axis=1, keepdims=True)
    filled = jnp.where(f2 == 0, bm, f2)
    row_ids = jnp.maximum(filled - 1, 0).astype(jnp.int32)
    return be, row_ids.reshape(-1)


def _sc_mesh():
    return plsc.VectorSubcoreMesh(core_axis_name="c", subcore_axis_name="s")


@functools.lru_cache(maxsize=None)
def _gather_kernel():
    @functools.partial(
        pl.kernel,
        mesh=_sc_mesh(),
        out_type=[jax.ShapeDtypeStruct((RS, D), _f32),
                  jax.ShapeDtypeStruct((RS, D), _f32),
                  jax.ShapeDtypeStruct((RS, ARP), _f32)],
        scratch_types=[pltpu.VMEM((RPW // 2,), jnp.int32),
                       pltpu.VMEM((RPW // 2, ARP), _f32),
                       pltpu.VMEM((CH,), jnp.int32),
                       pltpu.VMEM((CH,), jnp.int32),
                       pltpu.VMEM((CH, D), _f32),
                       pltpu.VMEM((CH, D), _f32),
                       pltpu.SemaphoreType.DMA,
                       pltpu.SemaphoreType.DMA,
                       pltpu.SemaphoreType.DMA],
    )
    def _sc_gather(s1_hbm, s2_hbm, ar_hbm, idx_hbm, o1, o2, o3,
                   ia, ba, i0, i1, b0, b1, sem0, sem1, sema):
        wid = lax.axis_index("s") * NC + lax.axis_index("c")
        base = wid * RPW
        HA = RPW // 2
        pltpu.sync_copy(idx_hbm.at[pl.ds(base, HA)], ia)
        ar_cp = pltpu.async_copy(ar_hbm.at[ia], ba, sema)

        idxs = (i0, i1)
        bufs = (b0, b1)
        sems = (sem0, sem1)
        NTOT = 2 * NCHH

        def src_dst_off(ci):
            if ci < NCHH:
                return s1_hbm, o1, base + ci * CH
            return s2_hbm, o2, base + (ci - NCHH) * CH

        def fire(ci):
            sl = ci % 2
            src, _, off = src_dst_off(ci)
            pltpu.sync_copy(idx_hbm.at[pl.ds(off, CH)], idxs[sl])
            return pltpu.async_copy(src.at[idxs[sl]], bufs[sl], sems[sl])

        pend = [fire(0), fire(1)]
        for ci in range(NTOT):
            sl = ci % 2
            _, dst, off = src_dst_off(ci)
            pend[sl].wait()
            pltpu.sync_copy(bufs[sl], dst.at[pl.ds(off, CH)])
            if ci + 2 < NTOT:
                pend[sl] = fire(ci + 2)

        ar_cp.wait()
        pltpu.sync_copy(ba, o3.at[pl.ds(base, HA)])
        pltpu.sync_copy(idx_hbm.at[pl.ds(base + HA, HA)], ia)
        pltpu.async_copy(ar_hbm.at[ia], ba, sema).wait()
        pltpu.sync_copy(ba, o3.at[pl.ds(base + HA, HA)])

    return _sc_gather


@functools.lru_cache(maxsize=None)
def _scatter_kernel():
    @functools.partial(
        pl.kernel,
        mesh=_sc_mesh(),
        out_type=[jax.ShapeDtypeStruct((B, D), _f32),
                  jax.ShapeDtypeStruct((B, D), _f32)],
        scratch_types=[pltpu.VMEM((CH,), jnp.int32),
                       pltpu.VMEM((CH,), jnp.int32),
                       pltpu.VMEM((CH, D), _f32),
                       pltpu.VMEM((CH, D), _f32),
                       pltpu.SemaphoreType.DMA,
                       pltpu.SemaphoreType.DMA],
    )
    def _sc_scatter(mu_s_hbm, lv_s_hbm, idx_hbm, mu_o, lv_o,
                    i0, i1, b0, b1, sem0, sem1):
        wid = lax.axis_index("s") * NC + lax.axis_index("c")
        base = wid * RPW
        idxs = (i0, i1)
        bufs = (b0, b1)
        sems = (sem0, sem1)
        NTOT = 2 * NCHH

        def src_dst_off(ci):
            if ci < NCHH:
                return mu_s_hbm, mu_o, base + ci * CH
            return lv_s_hbm, lv_o, base + (ci - NCHH) * CH

        pend = {}
        for ci in range(NTOT):
            sl = ci % 2
            src, dst, off = src_dst_off(ci)
            if sl in pend:
                pend[sl].wait()
            pltpu.sync_copy(idx_hbm.at[pl.ds(off, CH)], idxs[sl])
            pltpu.sync_copy(src.at[pl.ds(off, CH)], bufs[sl])
            pend[sl] = pltpu.async_copy(bufs[sl], dst.at[idxs[sl]], sems[sl])
        for c in pend.values():
            c.wait()

    return _sc_scatter


def _dot(x, w):
    return lax.dot_general(x, w, (((1,), (0,)), ((), ())),
                           preferred_element_type=_f32)


def _enc_body(be_ref, s1, s2, ar,
              se1W, se1b, se2W, se2b, nse1W, nse1b, nse2W, nse2b,
              Wh1, Wh2, AEp, REv, bp, ce2W, ce2b, muW, mub, lvW, lvb,
              mu_o, lv_o):
    t1 = jnp.maximum(_dot(s1[...].astype(_bf16), se1W[0]) + se1b[0],
                     0.0).astype(_bf16)
    h1 = (_dot(t1, se2W[0]) + se2b[0]).astype(_bf16)
    t2 = jnp.maximum(_dot(s2[...].astype(_bf16), nse1W[0]) + nse1b[0],
                     0.0).astype(_bf16)
    h2 = (_dot(t2, nse2W[0]) + nse2b[0]).astype(_bf16)
    ab = ar[:, :A].astype(_bf16)
    rb = ar[:, A:A + 1]
    cpre = (_dot(h1, Wh1[0]) + _dot(h2, Wh2[0]) + _dot(ab, AEp[0])
            + rb * REv[0] + bp[0])
    c2 = (_dot(jnp.maximum(cpre, 0.0).astype(_bf16), ce2W[0])
          + ce2b[0]).astype(_bf16)
    mu_o[...] = _dot(c2, muW[0]) + mub[0]
    lv_o[...] = _dot(c2, lvW[0]) + lvb[0]


def _softmax(x):
    m = jnp.max(x, axis=-1, keepdims=True)
    e = jnp.exp(x - m)
    return e / jnp.sum(e, axis=-1, keepdims=True)


def _dec_body(mu, lv, eps,
              sd1W, sd1b, sd2W, sd2b, sd3W, sd3b,
              nsd1W, nsd1b, nsd2W, nsd2b, nsd3W, nsd3b,
              ad1W, ad1b, ad2W, ad2b,
              sd_o, nsd_o, ad_o, z_o):
    z = mu[...] + eps[...] * jnp.exp(0.5 * lv[...])
    z_o[...] = z
    zb = z.astype(_bf16)

    def mlp3(w1, b1, w2, b2, w3, b3):
        h = jnp.maximum(_dot(zb, w1[...]) + b1[0], 0.0).astype(_bf16)
        h = jnp.maximum(_dot(h, w2[...]) + b2[0], 0.0).astype(_bf16)
        return _softmax(_dot(h, w3[...]) + b3[0])

    sd_o[...] = mlp3(sd1W, sd1b, sd2W, sd2b, sd3W, sd3b)
    nsd_o[...] = mlp3(nsd1W, nsd1b, nsd2W, nsd2b, nsd3W, nsd3b)
    ha = jnp.maximum(_dot(zb, ad1W[...]) + ad1b[0], 0.0).astype(_bf16)
    ad_o[...] = _softmax(_dot(ha, ad2W[...]) + ad2b[0])


def kernel(s_t_1, a, r, s_t_2, labels, params):
    be, flat_ids = _routing(labels)

    ar = jnp.concatenate(
        [a, r, jnp.zeros((B, ARP - A - 1), _f32)], axis=1)

    s1_s, s2_s, ar_s = _gather_kernel()(s_t_1, s_t_2, ar, flat_ids)

    enc = params['enc']
    def stkw(xs):
        return jnp.stack([x.astype(_bf16) for x in xs])
    def stkb(xs):
        return jnp.stack([x.reshape(1, -1) for x in xs])
    se1W, se1b = stkw([e['se1'][0] for e in enc]), stkb([e['se1'][1] for e in enc])
    se2W, se2b = stkw([e['se2'][0] for e in enc]), stkb([e['se2'][1] for e in enc])
    nse1W, nse1b = stkw([e['nse1'][0] for e in enc]), stkb([e['nse1'][1] for e in enc])
    nse2W, nse2b = stkw([e['nse2'][0] for e in enc]), stkb([e['nse2'][1] for e in enc])
    ce2W, ce2b = stkw([e['ce2'][0] for e in enc]), stkb([e['ce2'][1] for e in enc])
    muW, mub = stkw([e['mu'][0] for e in enc]), stkb([e['mu'][1] for e in enc])
    lvW, lvb = stkw([e['lv'][0] for e in enc]), stkb([e['lv'][1] for e in enc])

    Wh1 = stkw([e['ce1'][0][:H] for e in enc])
    Wh2 = stkw([e['ce1'][0][H + 12:] for e in enc])
    AEp = stkw([e['ae'][0] @ e['ce1'][0][H:H + 8] for e in enc])
    REv = stkb([e['re'][0] @ e['ce1'][0][H + 8:H + 12] for e in enc])
    bp = stkb([e['ce1'][1] + e['ae'][1] @ e['ce1'][0][H:H + 8]
               + e['re'][1] @ e['ce1'][0][H + 8:H + 12] for e in enc])

    wspec3 = lambda d1, d2: pl.BlockSpec((1, d1, d2), lambda j, be: (be[j], 0, 0))
    wspec2 = lambda d1: pl.BlockSpec((1, 1, d1), lambda j, be: (be[j], 0, 0))
    rspec = lambda d1: pl.BlockSpec((BLK, d1), lambda j, be: (j, 0))

    enc_spec = pltpu.PrefetchScalarGridSpec(
        num_scalar_prefetch=1,
        grid=(NBLK,),
        in_specs=[
            rspec(D), rspec(D), rspec(ARP),
            wspec3(D, 128), wspec2(128), wspec3(128, H), wspec2(H),
            wspec3(D, 128), wspec2(128), wspec3(128, H), wspec2(H),
            wspec3(H, H), wspec3(H, H), wspec3(A, H), wspec2(H), wspec2(H),
            wspec3(H, H), wspec2(H),
            wspec3(H, D), wspec2(D), wspec3(H, D), wspec2(D),
        ],
        out_specs=[rspec(D), rspec(D)],
    )
    mu_s, lv_s = pl.pallas_call(
        _enc_body,
        grid_spec=enc_spec,
        out_shape=[jax.ShapeDtypeStruct((RS, D), _f32),
                   jax.ShapeDtypeStruct((RS, D), _f32)],
        compiler_params=pltpu.CompilerParams(
            dimension_semantics=("arbitrary",)),
    )(be, s1_s, s2_s, ar_s,
      se1W, se1b, se2W, se2b, nse1W, nse1b, nse2W, nse2b,
      Wh1, Wh2, AEp, REv, bp, ce2W, ce2b, muW, mub, lvW, lvb)

    mu, lv = _scatter_kernel()(mu_s, lv_s, flat_ids)

    try:
        with jax.ensure_compile_time_eval():
            eps = jax.random.normal(jax.random.key(42), (B, D), dtype=_f32)
    except Exception:
        eps = jax.random.normal(jax.random.key(42), (B, D), dtype=_f32)

    RB = 256
    row = lambda d1: pl.BlockSpec((RB, d1), lambda j: (j, 0))
    w2 = lambda d1, d2: pl.BlockSpec((d1, d2), lambda j: (0, 0))
    bvec = lambda d1: pl.BlockSpec((1, d1), lambda j: (0, 0))
    p = params
    sd, nsd, ad, z = pl.pallas_call(
        _dec_body,
        grid=(B // RB,),
        in_specs=[
            row(D), row(D), row(D),
            w2(D, 128), bvec(128), w2(128, 128), bvec(128), w2(128, D), bvec(D),
            w2(D, 128), bvec(128), w2(128, 128), bvec(128), w2(128, D), bvec(D),
            w2(D, H), bvec(H), w2(H, A), bvec(A),
        ],
        out_specs=[row(D), row(D), row(A), row(D)],
        out_shape=[jax.ShapeDtypeStruct((B, D), _f32),
                   jax.ShapeDtypeStruct((B, D), _f32),
                   jax.ShapeDtypeStruct((B, A), _f32),
                   jax.ShapeDtypeStruct((B, D), _f32)],
        compiler_params=pltpu.CompilerParams(
            dimension_semantics=("arbitrary",)),
    )(mu, lv, eps,
      p['sd1'][0].astype(_bf16), p['sd1'][1].reshape(1, -1),
      p['sd2'][0].astype(_bf16), p['sd2'][1].reshape(1, -1),
      p['sd3'][0].astype(_bf16), p['sd3'][1].reshape(1, -1),
      p['nsd1'][0].astype(_bf16), p['nsd1'][1].reshape(1, -1),
      p['nsd2'][0].astype(_bf16), p['nsd2'][1].reshape(1, -1),
      p['nsd3'][0].astype(_bf16), p['nsd3'][1].reshape(1, -1),
      p['ad1'][0].astype(_bf16), p['ad1'][1].reshape(1, -1),
      p['ad2'][0].astype(_bf16), p['ad2'][1].reshape(1, -1))

    rd = jnp.ones((B, 1), _f32)

    return (sd, ad, rd, nsd, mu, lv, z)
